# baseline (speedup 1.0000x reference)
import functools

import jax
import jax.numpy as jnp
from jax import lax
from jax.experimental import pallas as pl
from jax.experimental.pallas import tpu as pltpu
from jax.experimental.pallas import tpu_sc as plsc

_NUM_CORES = 2
_NUM_SUBCORES = 16
_NUM_WORKERS = _NUM_CORES * _NUM_SUBCORES

_L = 16
_BT = 128
_SG = 4


def _make_gather(batch, seq, D):
    nbt = batch // _BT
    ndt = D // 8
    bt_per_w = nbt // _NUM_WORKERS
    ngrp = seq // _SG
    grp = _SG * _BT
    assert nbt % _NUM_WORKERS == 0 and seq % _SG == 0 and D % 8 == 0

    mesh = plsc.VectorSubcoreMesh(core_axis_name="c", subcore_axis_name="s")

    @functools.partial(
        pl.kernel,
        out_type=jax.ShapeDtypeStruct((seq, ndt, nbt, 8 * _BT), jnp.float32),
        mesh=mesh,
        scratch_types=[
            pltpu.VMEM((_BT, seq), jnp.int32),
            pltpu.VMEM((seq * _BT,), jnp.int32),
            [pltpu.VMEM((grp, D), jnp.float32) for _ in range(2)],
            [pltpu.VMEM((_SG * D * _BT,), jnp.float32) for _ in range(2)],
            [pltpu.SemaphoreType.DMA for _ in range(2)],
            [pltpu.SemaphoreType.DMA for _ in range(2)],
        ],
        compiler_params=pltpu.CompilerParams(
            use_tc_tiling_on_sc=False, needs_layout_passes=False),
    )
    def k(zs_hbm, table_hbm, out_hbm, zsp_v, zst_v, rows_v, tiles_v, gsem, wsem):
        wid = lax.axis_index("s") * _NUM_CORES + lax.axis_index("c")
        iota = lax.iota(jnp.int32, _L)
        iota_bt = iota * _BT
        rvec = [(iota + k) & (_L - 1) for k in range(_L)]
        wvec = [((iota + k) & (_L - 1)) * _BT + iota for k in range(_L)]

        def wait_gather(sl):
            pltpu.make_async_copy(table_hbm.at[pl.ds(0, grp)], rows_v[sl], gsem[sl]).wait()

        def wait_writes(sl):
            for _ in range(_SG * ndt):
                pltpu.make_async_copy(
                    out_hbm.at[0, 0, 0], tiles_v[sl].at[pl.ds(0, 8 * _BT)], wsem[sl]).wait()

        @pl.loop(0, bt_per_w)
        def _piece(piece):
            bt = wid * bt_per_w + piece
            b0 = bt * _BT

            pltpu.sync_copy(zs_hbm.at[pl.ds(b0, _BT), :], zsp_v)

            @pl.loop(0, seq)
            def _zst(s):
                for bg in range(_BT // _L):
                    v = plsc.load_gather(zsp_v, [bg * _L + iota, jnp.full((_L,), s, jnp.int32)])
                    zst_v[pl.ds(s * _BT + bg * _L, _L)] = v

            def start_gather(g, sl):
                pltpu.async_copy(
                    table_hbm.at[zst_v.at[pl.ds(g * grp, grp)]], rows_v[sl], gsem[sl])

            def transpose(sl):
                @plsc.parallel_loop(0, _SG * _BT, step=_L, unroll=4)
                def _t(r0):
                    sg = r0 // _BT
                    base_w = sg * ((D - 1) * _BT) + r0
                    rows = r0 + iota
                    for dh in range(D // _L):
                        wb = base_w + dh * (_L * _BT)
                        for kk in range(_L):
                            v = plsc.load_gather(
                                rows_v[sl], [rows, rvec[kk] + dh * _L])
                            plsc.store_scatter(tiles_v[sl], [wvec[kk] + wb], v)

            def start_writes(g, sl):
                for sg in range(_SG):
                    for dt in range(ndt):
                        pltpu.async_copy(
                            tiles_v[sl].at[pl.ds(sg * (D * _BT) + dt * (8 * _BT), 8 * _BT)],
                            out_hbm.at[g * _SG + sg, dt, bt],
                            wsem[sl])

            start_gather(0, 0)

            @pl.loop(0, ngrp, step=2)
            def _grp(g0):
                for sl in (0, 1):
                    g = g0 + sl

                    @pl.when(g < ngrp - 1)
                    def _():
                        start_gather(g + 1, 1 - sl)

                    wait_gather(sl)

                    @pl.when(g >= 2)
                    def _():
                        wait_writes(sl)

                    transpose(sl)
                    start_writes(g, sl)

            for sl in (0, 1):
                wait_writes(sl)

    return k


def kernel(zs, emb_weight):
    batch, seq = zs.shape
    d = emb_weight.shape[1]
    out5 = _make_gather(batch, seq, d)(zs.astype(jnp.int32), emb_weight)
    out = out5.reshape(seq, d // 8, batch // 128, 8, 128)
    return jnp.transpose(out, (2, 4, 0, 1, 3)).reshape(batch, seq, d)


# speedup vs baseline: 1.5844x; 1.5844x over previous
import functools

import jax
import jax.numpy as jnp
from jax import lax
from jax.experimental import pallas as pl
from jax.experimental.pallas import tpu as pltpu
from jax.experimental.pallas import tpu_sc as plsc

_NUM_CORES = 2
_NUM_SUBCORES = 16
_NUM_WORKERS = _NUM_CORES * _NUM_SUBCORES

_L = 16
_BT = 128
_SG = 4


def _make_gather(batch, seq, D):
    nbt = batch // _BT
    ndt = D // 8
    bt_per_w = nbt // _NUM_WORKERS
    ngrp = seq // _SG
    grp = _SG * _BT
    assert nbt % _NUM_WORKERS == 0 and seq % _SG == 0 and D % 8 == 0

    mesh = plsc.VectorSubcoreMesh(core_axis_name="c", subcore_axis_name="s")

    @functools.partial(
        pl.kernel,
        out_type=jax.ShapeDtypeStruct((seq, ndt, nbt, 8 * _BT), jnp.float32),
        mesh=mesh,
        scratch_types=[
            pltpu.VMEM((_BT, seq), jnp.int32),
            pltpu.VMEM((seq * _BT,), jnp.int32),
            [pltpu.VMEM((grp, D), jnp.float32) for _ in range(2)],
            [pltpu.VMEM((_SG * D * _BT,), jnp.float32) for _ in range(2)],
            [pltpu.SemaphoreType.DMA for _ in range(2)],
            [pltpu.SemaphoreType.DMA for _ in range(2)],
        ],
        compiler_params=pltpu.CompilerParams(
            use_tc_tiling_on_sc=False, needs_layout_passes=False),
    )
    def k(zs_hbm, table_hbm, out_hbm, zsp_v, zst_v, rows_v, tiles_v, gsem, wsem):
        wid = lax.axis_index("s") * _NUM_CORES + lax.axis_index("c")
        iota = lax.iota(jnp.int32, _L)
        iota_bt = iota * _BT
        rvec = [(iota + k) & (_L - 1) for k in range(_L)]
        wvec = [((iota + k) & (_L - 1)) * _BT + iota for k in range(_L)]

        def wait_gather(sl):
            pltpu.make_async_copy(table_hbm.at[pl.ds(0, grp)], rows_v[sl], gsem[sl]).wait()

        def wait_writes(sl):
            for _ in range(_SG * ndt):
                pltpu.make_async_copy(
                    out_hbm.at[0, 0, 0], tiles_v[sl].at[pl.ds(0, 8 * _BT)], wsem[sl]).wait()

        @pl.loop(0, bt_per_w)
        def _piece(piece):
            bt = wid * bt_per_w + piece
            b0 = bt * _BT

            pltpu.sync_copy(zs_hbm.at[pl.ds(b0, _BT), :], zsp_v)

            @pl.loop(0, seq)
            def _zst(s):
                for bg in range(_BT // _L):
                    v = plsc.load_gather(zsp_v, [bg * _L + iota, jnp.full((_L,), s, jnp.int32)])
                    zst_v[pl.ds(s * _BT + bg * _L, _L)] = v

            def start_gather(g, sl):
                pltpu.async_copy(
                    table_hbm.at[zst_v.at[pl.ds(g * grp, grp)]], rows_v[sl], gsem[sl])

            def transpose(sl):
                @plsc.parallel_loop(0, _SG * _BT, step=_L, unroll=1)
                def _t(r0):
                    sg = r0 // _BT
                    base_w = sg * ((D - 1) * _BT) + r0
                    rows = r0 + iota
                    for dh in range(D // _L):
                        wb = base_w + dh * (_L * _BT)
                        for kk in range(_L):
                            v = plsc.load_gather(
                                rows_v[sl], [rows, rvec[kk] + dh * _L])
                            plsc.store_scatter(tiles_v[sl], [wvec[kk] + wb], v)

            def start_writes(g, sl):
                for sg in range(_SG):
                    for dt in range(ndt):
                        pltpu.async_copy(
                            tiles_v[sl].at[pl.ds(sg * (D * _BT) + dt * (8 * _BT), 8 * _BT)],
                            out_hbm.at[g * _SG + sg, dt, bt],
                            wsem[sl])

            start_gather(0, 0)

            @pl.loop(0, ngrp, step=2)
            def _grp(g0):
                for sl in (0, 1):
                    g = g0 + sl

                    @pl.when(g < ngrp - 1)
                    def _():
                        start_gather(g + 1, 1 - sl)

                    wait_gather(sl)

                    @pl.when(g >= 2)
                    def _():
                        wait_writes(sl)

                    transpose(sl)
                    start_writes(g, sl)

            for sl in (0, 1):
                wait_writes(sl)

    return k


def kernel(zs, emb_weight):
    batch, seq = zs.shape
    d = emb_weight.shape[1]
    out5 = _make_gather(batch, seq, d)(zs.astype(jnp.int32), emb_weight)
    out = out5.reshape(seq, d // 8, batch // 128, 8, 128)
    return jnp.transpose(out, (2, 4, 0, 1, 3)).reshape(batch, seq, d)


# speedup vs baseline: 1.6871x; 1.0648x over previous
import functools

import jax
import jax.numpy as jnp
from jax import lax
from jax.experimental import pallas as pl
from jax.experimental.pallas import tpu as pltpu
from jax.experimental.pallas import tpu_sc as plsc

_NUM_CORES = 2
_NUM_SUBCORES = 16
_NUM_WORKERS = _NUM_CORES * _NUM_SUBCORES

_L = 16
_BT = 128
_SG = 4


def _make_gather(batch, seq, D):
    nbt = batch // _BT
    ndt = D // 8
    bt_per_w = nbt // _NUM_WORKERS
    ngrp = seq // _SG
    grp = _SG * _BT
    assert nbt % _NUM_WORKERS == 0 and seq % _SG == 0 and D % 8 == 0

    mesh = plsc.VectorSubcoreMesh(core_axis_name="c", subcore_axis_name="s")

    @functools.partial(
        pl.kernel,
        out_type=jax.ShapeDtypeStruct((seq, ndt, nbt, 8 * _BT), jnp.float32),
        mesh=mesh,
        scratch_types=[
            pltpu.VMEM((_BT, seq), jnp.int32),
            pltpu.VMEM((seq * _BT,), jnp.int32),
            [pltpu.VMEM((grp, D), jnp.float32) for _ in range(2)],
            [pltpu.VMEM((_SG * D * _BT,), jnp.float32) for _ in range(2)],
            [pltpu.SemaphoreType.DMA for _ in range(2)],
            [pltpu.SemaphoreType.DMA for _ in range(2)],
        ],
        compiler_params=pltpu.CompilerParams(
            use_tc_tiling_on_sc=False, needs_layout_passes=False),
    )
    def k(zs_hbm, table_hbm, out_hbm, zsp_v, zst_v, rows_v, tiles_v, gsem, wsem):
        wid = lax.axis_index("s") * _NUM_CORES + lax.axis_index("c")
        iota = lax.iota(jnp.int32, _L)
        iota_bt = iota * _BT
        rvec = [(iota + k) & (_L - 1) for k in range(_L)]
        wvec = [((iota + k) & (_L - 1)) * _BT + iota for k in range(_L)]

        def wait_gather(sl):
            pltpu.make_async_copy(table_hbm.at[pl.ds(0, grp)], rows_v[sl], gsem[sl]).wait()

        def wait_writes(sl):
            for _ in range(_SG * ndt):
                pltpu.make_async_copy(
                    out_hbm.at[0, 0, 0], tiles_v[sl].at[pl.ds(0, 8 * _BT)], wsem[sl]).wait()

        @pl.loop(0, bt_per_w)
        def _piece(piece):
            bt = wid * bt_per_w + piece
            b0 = bt * _BT

            pltpu.sync_copy(zs_hbm.at[pl.ds(b0, _BT), :], zsp_v)

            @plsc.parallel_loop(0, seq)
            def _zst(s):
                for bg in range(_BT // _L):
                    v = plsc.load_gather(zsp_v, [bg * _L + iota, jnp.full((_L,), s, jnp.int32)])
                    zst_v[pl.ds(s * _BT + bg * _L, _L)] = v

            def start_gather(g, sl):
                pltpu.async_copy(
                    table_hbm.at[zst_v.at[pl.ds(g * grp, grp)]], rows_v[sl], gsem[sl])

            def transpose(sl):
                @plsc.parallel_loop(0, _SG * _BT, step=_L, unroll=1)
                def _t(r0):
                    sg = r0 // _BT
                    base_w = sg * ((D - 1) * _BT) + r0
                    rows = r0 + iota
                    for dh in range(D // _L):
                        wb = base_w + dh * (_L * _BT)
                        for kk in range(_L):
                            v = plsc.load_gather(
                                rows_v[sl], [rows, rvec[kk] + dh * _L])
                            plsc.store_scatter(tiles_v[sl], [wvec[kk] + wb], v)

            def start_writes(g, sl):
                for sg in range(_SG):
                    for dt in range(ndt):
                        pltpu.async_copy(
                            tiles_v[sl].at[pl.ds(sg * (D * _BT) + dt * (8 * _BT), 8 * _BT)],
                            out_hbm.at[g * _SG + sg, dt, bt],
                            wsem[sl])

            start_gather(0, 0)

            @pl.loop(0, ngrp, step=2)
            def _grp(g0):
                for sl in (0, 1):
                    g = g0 + sl

                    @pl.when(g < ngrp - 1)
                    def _():
                        start_gather(g + 1, 1 - sl)

                    wait_gather(sl)

                    @pl.when(g >= 2)
                    def _():
                        wait_writes(sl)

                    transpose(sl)
                    start_writes(g, sl)

            for sl in (0, 1):
                wait_writes(sl)

    return k


def kernel(zs, emb_weight):
    batch, seq = zs.shape
    d = emb_weight.shape[1]
    out5 = _make_gather(batch, seq, d)(zs.astype(jnp.int32), emb_weight)
    out = out5.reshape(seq, d // 8, batch // 128, 8, 128)
    return jnp.transpose(out, (2, 4, 0, 1, 3)).reshape(batch, seq, d)


# speedup vs baseline: 1.6907x; 1.0021x over previous
import functools

import jax
import jax.numpy as jnp
from jax import lax
from jax.experimental import pallas as pl
from jax.experimental.pallas import tpu as pltpu
from jax.experimental.pallas import tpu_sc as plsc

_NUM_CORES = 2
_NUM_SUBCORES = 16
_NUM_WORKERS = _NUM_CORES * _NUM_SUBCORES

_L = 16
_BT = 128
_SG = 4


def _make_gather(batch, seq, D):
    nbt = batch // _BT
    ndt = D // 8
    bt_per_w = nbt // _NUM_WORKERS
    ngrp = seq // _SG
    grp = _SG * _BT
    assert nbt % _NUM_WORKERS == 0 and seq % _SG == 0 and D % 8 == 0

    mesh = plsc.VectorSubcoreMesh(core_axis_name="c", subcore_axis_name="s")

    @functools.partial(
        pl.kernel,
        out_type=jax.ShapeDtypeStruct((seq, ndt, nbt, 8 * _BT), jnp.float32),
        mesh=mesh,
        scratch_types=[
            pltpu.VMEM((_BT, seq), jnp.int32),
            pltpu.VMEM((seq * _BT,), jnp.int32),
            [pltpu.VMEM((grp, D), jnp.float32) for _ in range(2)],
            [pltpu.VMEM((_SG * D * _BT,), jnp.float32) for _ in range(2)],
            [pltpu.SemaphoreType.DMA for _ in range(2)],
            [pltpu.SemaphoreType.DMA for _ in range(2)],
        ],
        compiler_params=pltpu.CompilerParams(
            use_tc_tiling_on_sc=False, needs_layout_passes=False),
    )
    def k(zs_hbm, table_hbm, out_hbm, zsp_v, zst_v, rows_v, tiles_v, gsem, wsem):
        wid = lax.axis_index("s") * _NUM_CORES + lax.axis_index("c")
        iota = lax.iota(jnp.int32, _L)
        rvec = [(iota + k) & (_L - 1) for k in range(_L)]
        wvec = [((iota + k) & (_L - 1)) * _BT + iota for k in range(_L)]

        def wait_gather(sl):
            pltpu.make_async_copy(table_hbm.at[pl.ds(0, grp)], rows_v[sl], gsem[sl]).wait()

        def wait_writes(sl):
            for _ in range(_SG * ndt):
                pltpu.make_async_copy(
                    out_hbm.at[0, 0, 0], tiles_v[sl].at[pl.ds(0, 8 * _BT)], wsem[sl]).wait()

        @pl.loop(0, bt_per_w)
        def _piece(piece):
            bt = wid * bt_per_w + piece
            b0 = bt * _BT

            pltpu.sync_copy(zs_hbm.at[pl.ds(b0, _BT), :], zsp_v)

            @plsc.parallel_loop(0, seq)
            def _zst(s):
                for bg in range(_BT // _L):
                    v = plsc.load_gather(zsp_v, [bg * _L + iota, jnp.full((_L,), s, jnp.int32)])
                    zst_v[pl.ds(s * _BT + bg * _L, _L)] = v

            def start_gather(g, sl):
                pltpu.async_copy(
                    table_hbm.at[zst_v.at[pl.ds(g * grp, grp)]], rows_v[sl], gsem[sl])

            def transpose(sl):
                @plsc.parallel_loop(0, _SG * _BT, step=_L, unroll=1)
                def _t(r0):
                    sg = r0 // _BT
                    base_w = sg * ((D - 1) * _BT) + r0
                    rows = r0 + iota
                    for dh in range(D // _L):
                        wb = base_w + dh * (_L * _BT)
                        for kk in range(_L):
                            v = plsc.load_gather(
                                rows_v[sl], [rows, rvec[kk] + dh * _L])
                            plsc.store_scatter(tiles_v[sl], [wvec[kk] + wb], v)

            def start_writes(g, sl):
                for sg in range(_SG):
                    for dt in range(ndt):
                        pltpu.async_copy(
                            tiles_v[sl].at[pl.ds(sg * (D * _BT) + dt * (8 * _BT), 8 * _BT)],
                            out_hbm.at[g * _SG + sg, dt, bt],
                            wsem[sl])

            start_gather(0, 0)

            @pl.loop(0, ngrp, step=2)
            def _grp(g0):
                for sl in (0, 1):
                    g = g0 + sl

                    @pl.when(g < ngrp - 1)
                    def _():
                        start_gather(g + 1, 1 - sl)

                    wait_gather(sl)

                    @pl.when(g >= 2)
                    def _():
                        wait_writes(sl)

                    transpose(sl)
                    start_writes(g, sl)

            for sl in (0, 1):
                wait_writes(sl)

    return k


def kernel(zs, emb_weight):
    batch, seq = zs.shape
    d = emb_weight.shape[1]
    out5 = _make_gather(batch, seq, d)(zs.astype(jnp.int32), emb_weight)
    out = out5.reshape(seq, d // 8, batch // 128, 8, 128)
    return jnp.transpose(out, (2, 4, 0, 1, 3)).reshape(batch, seq, d)
